# Initial kernel scaffold; baseline (speedup 1.0000x reference)
import functools

import jax
import jax.numpy as jnp
from jax import lax
from jax.experimental import pallas as pl
from jax.experimental.pallas import tpu as pltpu
from jax.experimental.pallas import tpu_sc as plsc

_D = 49152
_B = 1024
_NC = 2
_NS = 16
_NW = _NC * _NS
_BPW = _B // _NW


def _gather_body(table_hbm, idx_hbm, out_hbm, idx_v, buf0, buf1,
                 gsem0, gsem1, ssem0, ssem1):
    wid = lax.axis_index("s") * _NC + lax.axis_index("c")
    base = wid * _BPW
    pltpu.sync_copy(idx_hbm.at[pl.ds(base, _BPW)], idx_v)

    bufs = (buf0, buf1)
    gsems = (gsem0, gsem1)
    ssems = (ssem0, ssem1)

    gathers = [None, None]
    stores = [None, None]

    gathers[0] = pltpu.make_async_copy(
        table_hbm.at[idx_v.at[pl.ds(0, 1)]], bufs[0], gsems[0])
    gathers[0].start()

    for j in range(_BPW):
        cur = j % 2
        nxt = (j + 1) % 2
        if j + 1 < _BPW:
            if stores[nxt] is not None:
                stores[nxt].wait()
            gathers[nxt] = pltpu.make_async_copy(
                table_hbm.at[idx_v.at[pl.ds(j + 1, 1)]], bufs[nxt],
                gsems[nxt])
            gathers[nxt].start()
        gathers[cur].wait()
        stores[cur] = pltpu.make_async_copy(
            bufs[cur], out_hbm.at[pl.ds(base + j, 1)], ssems[cur])
        stores[cur].start()

    for s in stores:
        if s is not None:
            s.wait()


@jax.jit
def _gather(table, idx):
    mesh = plsc.VectorSubcoreMesh(core_axis_name="c", subcore_axis_name="s")
    f = pl.kernel(
        _gather_body,
        out_type=jax.ShapeDtypeStruct((_B, _D), jnp.float32),
        mesh=mesh,
        scratch_types=[
            pltpu.VMEM((_BPW,), jnp.int32),
            pltpu.VMEM((1, _D), jnp.float32),
            pltpu.VMEM((1, _D), jnp.float32),
            pltpu.SemaphoreType.DMA,
            pltpu.SemaphoreType.DMA,
            pltpu.SemaphoreType.DMA,
            pltpu.SemaphoreType.DMA,
        ],
    )
    return f(table, idx)


def kernel(prefix, embedding):
    idx = prefix.reshape(-1).astype(jnp.int32)
    out = _gather(embedding, idx)
    return out.reshape(prefix.shape[0], prefix.shape[1], _D)


# speedup vs baseline: 2.1902x; 2.1902x over previous
import functools

import jax
import jax.numpy as jnp
from jax import lax
from jax.experimental import pallas as pl
from jax.experimental.pallas import tpu as pltpu
from jax.experimental.pallas import tpu_sc as plsc

_D = 49152
_B = 1024
_NC = 2
_NS = 16
_NW = _NC * _NS
_BPW = _B // _NW


def _gather_body(table_hbm, idx_hbm, out_hbm, idx_v, buf0, buf1,
                 gsem0, gsem1, ssem0, ssem1):
    wid = lax.axis_index("s") * _NC + lax.axis_index("c")
    base = wid * _BPW
    pltpu.sync_copy(idx_hbm.at[pl.ds(base, _BPW)], idx_v)

    bufs = (buf0, buf1)
    gsems = (gsem0, gsem1)
    ssems = (ssem0, ssem1)

    gathers = [None, None]
    stores = [None, None]

    gathers[0] = pltpu.make_async_copy(
        table_hbm.at[idx_v.at[0]], bufs[0], gsems[0])
    gathers[0].start()

    for j in range(_BPW):
        cur = j % 2
        nxt = (j + 1) % 2
        if j + 1 < _BPW:
            if stores[nxt] is not None:
                stores[nxt].wait()
            gathers[nxt] = pltpu.make_async_copy(
                table_hbm.at[idx_v.at[j + 1]], bufs[nxt],
                gsems[nxt])
            gathers[nxt].start()
        gathers[cur].wait()
        stores[cur] = pltpu.make_async_copy(
            bufs[cur], out_hbm.at[pl.ds(base + j, 1)], ssems[cur])
        stores[cur].start()

    for s in stores:
        if s is not None:
            s.wait()


@jax.jit
def _gather(table, idx):
    mesh = plsc.VectorSubcoreMesh(core_axis_name="c", subcore_axis_name="s")
    f = pl.kernel(
        _gather_body,
        out_type=jax.ShapeDtypeStruct((_B, _D), jnp.float32),
        mesh=mesh,
        scratch_types=[
            pltpu.VMEM((_BPW, 1), jnp.int32),
            pltpu.VMEM((1, _D), jnp.float32),
            pltpu.VMEM((1, _D), jnp.float32),
            pltpu.SemaphoreType.DMA,
            pltpu.SemaphoreType.DMA,
            pltpu.SemaphoreType.DMA,
            pltpu.SemaphoreType.DMA,
        ],
    )
    return f(table, idx)


def kernel(prefix, embedding):
    idx = prefix.reshape(-1, 1).astype(jnp.int32)
    out = _gather(embedding, idx)
    return out.reshape(prefix.shape[0], prefix.shape[1], _D)


# speedup vs baseline: 3.0418x; 1.3888x over previous
import functools

import jax
import jax.numpy as jnp
from jax import lax
from jax.experimental import pallas as pl
from jax.experimental.pallas import tpu as pltpu
from jax.experimental.pallas import tpu_sc as plsc

_V = 128
_D = 49152
_B = 1024
_NC = 2
_NS = 16
_NW = _NC * _NS
_BPW = _B // _NW
_C = 1536
_NCHUNK = _D // _C
_RPT = _V // _NS
_L = 16


def _gather_body(table_hbm, idx_hbm, out_hbm,
                 idx_v, rbuf0, rbuf1, sbuf0, sbuf1,
                 lsem0, lsem1, gsem0, gsem1, ssem0, ssem1):
    cid = lax.axis_index("c")
    sid = lax.axis_index("s")
    wid = sid * _NC + cid
    base = wid * _BPW
    pltpu.sync_copy(idx_hbm.at[pl.ds(base, _BPW)], idx_v)

    scalars = []
    for v in range(_BPW // _L):
        vec = idx_v[pl.ds(v * _L, _L)]
        for j in range(_L):
            scalars.append(vec[j])

    rbufs = (rbuf0, rbuf1)
    sbufs = (sbuf0, sbuf1)
    lsems = (lsem0, lsem1)
    gsems = (gsem0, gsem1)
    ssems = (ssem0, ssem1)
    row0 = sid * _RPT

    def load_desc(c, slot):
        off = pl.multiple_of(c * _C, _C)
        return pltpu.make_async_copy(
            table_hbm.at[pl.ds(row0, _RPT), pl.ds(off, _C)],
            sbufs[slot].at[pl.ds(row0, _RPT)],
            lsems[slot])

    def store_desc(c, slot):
        off = pl.multiple_of(c * _C, _C)
        return pltpu.make_async_copy(
            rbufs[slot],
            out_hbm.at[pl.ds(base, _BPW), pl.ds(off, _C)],
            ssems[slot])

    def do_chunk(c, slot):
        load_desc(c, slot).wait()
        plsc.subcore_barrier()

        @pl.when(c + 1 < _NCHUNK)
        def _():
            load_desc(c + 1, 1 - slot).start()

        @pl.when(c >= 2)
        def _():
            store_desc(c - 2, slot).wait()

        copies = []
        for j in range(_BPW):
            cp = pltpu.make_async_copy(
                sbufs[slot].at[pl.ds(scalars[j], 1)],
                rbufs[slot].at[pl.ds(j, 1)],
                gsems[slot])
            cp.start()
            copies.append(cp)
        for cp in copies:
            cp.wait()
        store_desc(c, slot).start()

    load_desc(0, 0).start()

    def pair_body(p, carry):
        do_chunk(p * 2, 0)
        do_chunk(p * 2 + 1, 1)
        return carry

    lax.fori_loop(0, _NCHUNK // 2, pair_body, 0)

    store_desc(_NCHUNK - 2, 0).wait()
    store_desc(_NCHUNK - 1, 1).wait()


@jax.jit
def _gather(table, idx):
    mesh = plsc.VectorSubcoreMesh(core_axis_name="c", subcore_axis_name="s")
    f = pl.kernel(
        _gather_body,
        out_type=jax.ShapeDtypeStruct((_B, _D), jnp.float32),
        mesh=mesh,
        scratch_types=[
            pltpu.VMEM((_BPW,), jnp.int32),
            pltpu.VMEM((_BPW, _C), jnp.float32),
            pltpu.VMEM((_BPW, _C), jnp.float32),
            pltpu.VMEM_SHARED((_V, _C), jnp.float32),
            pltpu.VMEM_SHARED((_V, _C), jnp.float32),
            pltpu.SemaphoreType.DMA,
            pltpu.SemaphoreType.DMA,
            pltpu.SemaphoreType.DMA,
            pltpu.SemaphoreType.DMA,
            pltpu.SemaphoreType.DMA,
            pltpu.SemaphoreType.DMA,
        ],
    )
    return f(table, idx)


def kernel(prefix, embedding):
    idx = prefix.reshape(-1).astype(jnp.int32)
    out = _gather(embedding, idx)
    return out.reshape(prefix.shape[0], prefix.shape[1], _D)
